# Initial kernel scaffold; baseline (speedup 1.0000x reference)
import functools
import jax
import jax.numpy as jnp
from jax import lax
from jax.experimental import pallas as pl
from jax.experimental.pallas import tpu as pltpu
from jax.experimental.pallas import tpu_sc as plsc

N_PAD = 10240
NC = 2
NS = 16
NW = NC * NS
CHUNK = 128
ROWS_PER_TILE = N_PAD // NS


def _deg_body(nch, dst_hbm, zeros_hbm, out_hbm, idx_v, ones_v, dacc, sem):
    c = lax.axis_index("c")
    s = lax.axis_index("s")
    wid = s * NC + c
    for k in range(CHUNK // 16):
        ones_v[pl.ds(k * 16, 16)] = jnp.ones((16,), jnp.float32)
    pltpu.sync_copy(zeros_hbm.at[pl.ds(s * ROWS_PER_TILE, ROWS_PER_TILE)],
                    dacc.at[pl.ds(s * ROWS_PER_TILE, ROWS_PER_TILE)])
    pltpu.sync_copy(dst_hbm.at[wid], idx_v)
    plsc.subcore_barrier()
    for j in range(nch):
        pltpu.sync_copy(ones_v, dacc.at[idx_v.at[j]], add=True)
    plsc.subcore_barrier()
    pltpu.sync_copy(dacc.at[pl.ds(s * ROWS_PER_TILE, ROWS_PER_TILE)],
                    out_hbm.at[c, pl.ds(s * ROWS_PER_TILE, ROWS_PER_TILE)])


def _agg_body(nch, featn_hbm, src_hbm, dst_hbm, zeros_hbm, out_hbm,
              sidx_v, didx_v, rows0, rows1, acc, sem0, sem1):
    c = lax.axis_index("c")
    s = lax.axis_index("s")
    wid = s * NC + c
    pltpu.sync_copy(zeros_hbm.at[pl.ds(s * ROWS_PER_TILE, ROWS_PER_TILE)],
                    acc.at[pl.ds(s * ROWS_PER_TILE, ROWS_PER_TILE)])
    pltpu.sync_copy(src_hbm.at[wid], sidx_v)
    pltpu.sync_copy(dst_hbm.at[wid], didx_v)
    plsc.subcore_barrier()
    bufs = (rows0, rows1)
    sems = (sem0, sem1)
    handles = [pltpu.async_copy(featn_hbm.at[sidx_v.at[0]], bufs[0], sems[0])]
    for j in range(nch):
        if j + 1 < nch:
            handles.append(pltpu.async_copy(
                featn_hbm.at[sidx_v.at[j + 1]],
                bufs[(j + 1) % 2], sems[(j + 1) % 2]))
        handles[j].wait()
        pltpu.sync_copy(bufs[j % 2], acc.at[didx_v.at[j]], add=True)
    plsc.subcore_barrier()
    pltpu.sync_copy(acc.at[pl.ds(s * ROWS_PER_TILE, ROWS_PER_TILE)],
                    out_hbm.at[c, pl.ds(s * ROWS_PER_TILE, ROWS_PER_TILE)])


def _norm_scale_body(deg_ref, feat_ref, featn_ref, norm_ref):
    d = deg_ref[0] + deg_ref[1]
    norm = lax.rsqrt(jnp.maximum(d, 1.0))
    norm_ref[...] = norm
    featn_ref[...] = feat_ref[...] * norm


def _out_body(acc_ref, w_ref, norm_ref, bias_ref, out_ref):
    a = acc_ref[0] + acc_ref[1]
    y = jnp.dot(a, w_ref[...], preferred_element_type=jnp.float32)
    out_ref[...] = y * norm_ref[...] + bias_ref[...]


def kernel(feat, edge_index, weight, bias):
    n, d_in = feat.shape
    d_out = weight.shape[1]
    e = edge_index.shape[1]
    nch = -(-e // (NW * CHUNK))
    e_pad = NW * CHUNK * nch

    src = edge_index[0].astype(jnp.int32)
    dst = edge_index[1].astype(jnp.int32)
    pad = jnp.full((e_pad - e,), n, jnp.int32)
    src3 = jnp.concatenate([src, pad]).reshape(NW, nch, CHUNK)
    dst3 = jnp.concatenate([dst, pad]).reshape(NW, nch, CHUNK)

    feat_pad = jnp.zeros((N_PAD, d_in), jnp.float32).at[:n].set(feat)
    zeros2d = jnp.zeros((N_PAD, d_in), jnp.float32)
    zeros1d = jnp.zeros((N_PAD,), jnp.float32)

    mesh = plsc.VectorSubcoreMesh(core_axis_name="c", subcore_axis_name="s")

    deg2 = pl.kernel(
        functools.partial(_deg_body, nch),
        out_type=jax.ShapeDtypeStruct((NC, N_PAD), jnp.float32),
        mesh=mesh,
        scratch_types=[
            pltpu.VMEM((nch, CHUNK), jnp.int32),
            pltpu.VMEM((CHUNK,), jnp.float32),
            pltpu.VMEM_SHARED((N_PAD,), jnp.float32),
            pltpu.SemaphoreType.DMA,
        ],
    )(dst3, zeros1d)

    deg2 = deg2.reshape(NC, N_PAD, 1)

    blk = 1280
    grid = N_PAD // blk
    featn, norm = pl.pallas_call(
        _norm_scale_body,
        grid=(grid,),
        in_specs=[
            pl.BlockSpec((NC, blk, 1), lambda i: (0, i, 0)),
            pl.BlockSpec((blk, d_in), lambda i: (i, 0)),
        ],
        out_specs=[
            pl.BlockSpec((blk, d_in), lambda i: (i, 0)),
            pl.BlockSpec((blk, 1), lambda i: (i, 0)),
        ],
        out_shape=[
            jax.ShapeDtypeStruct((N_PAD, d_in), jnp.float32),
            jax.ShapeDtypeStruct((N_PAD, 1), jnp.float32),
        ],
    )(deg2, feat_pad)

    acc2 = pl.kernel(
        functools.partial(_agg_body, nch),
        out_type=jax.ShapeDtypeStruct((NC, N_PAD, d_in), jnp.float32),
        mesh=mesh,
        scratch_types=[
            pltpu.VMEM((nch, CHUNK), jnp.int32),
            pltpu.VMEM((nch, CHUNK), jnp.int32),
            pltpu.VMEM((CHUNK, d_in), jnp.float32),
            pltpu.VMEM((CHUNK, d_in), jnp.float32),
            pltpu.VMEM_SHARED((N_PAD, d_in), jnp.float32),
            pltpu.SemaphoreType.DMA,
            pltpu.SemaphoreType.DMA,
        ],
    )(featn, src3, dst3, zeros2d)

    out = pl.pallas_call(
        _out_body,
        grid=(grid,),
        in_specs=[
            pl.BlockSpec((NC, blk, d_in), lambda i: (0, i, 0)),
            pl.BlockSpec((d_in, d_out), lambda i: (0, 0)),
            pl.BlockSpec((blk, 1), lambda i: (i, 0)),
            pl.BlockSpec((1, d_out), lambda i: (0, 0)),
        ],
        out_specs=pl.BlockSpec((blk, d_out), lambda i: (i, 0)),
        out_shape=jax.ShapeDtypeStruct((N_PAD, d_out), jnp.float32),
    )(acc2, weight, norm, bias.reshape(1, d_out))

    return out[:n]


# speedup vs baseline: 5.5974x; 5.5974x over previous
import functools
import jax
import jax.numpy as jnp
from jax import lax
from jax.experimental import pallas as pl
from jax.experimental.pallas import tpu as pltpu
from jax.experimental.pallas import tpu_sc as plsc

N_PAD = 10240
NC = 2
NS = 16
NW = NC * NS
CHUNK = 128
ROWS_PER_TILE = N_PAD // NS


def _deg_body(nch, dst_hbm, zeros_hbm, out_hbm, idx_v, ones_v, dacc, sem):
    c = lax.axis_index("c")
    s = lax.axis_index("s")
    wid = s * NC + c
    for k in range(CHUNK // 16):
        ones_v[pl.ds(k * 16, 16)] = jnp.ones((16,), jnp.float32)
    pltpu.sync_copy(zeros_hbm.at[pl.ds(s * ROWS_PER_TILE, ROWS_PER_TILE)],
                    dacc.at[pl.ds(s * ROWS_PER_TILE, ROWS_PER_TILE)])
    pltpu.sync_copy(dst_hbm.at[wid], idx_v)
    plsc.subcore_barrier()
    for j in range(nch):
        pltpu.sync_copy(ones_v, dacc.at[idx_v.at[j]], add=True)
    plsc.subcore_barrier()
    pltpu.sync_copy(dacc.at[pl.ds(s * ROWS_PER_TILE, ROWS_PER_TILE)],
                    out_hbm.at[c, pl.ds(s * ROWS_PER_TILE, ROWS_PER_TILE)])


IDX_BLK = 16


def _agg_body(nch, featn_hbm, src_hbm, dst_hbm, zeros_hbm, out_hbm,
              sidx_v, didx_v, rows0, rows1, acc, sem0, sem1):
    c = lax.axis_index("c")
    s = lax.axis_index("s")
    wid = s * NC + c
    pltpu.sync_copy(zeros_hbm.at[pl.ds(s * ROWS_PER_TILE, ROWS_PER_TILE)],
                    acc.at[pl.ds(s * ROWS_PER_TILE, ROWS_PER_TILE)])
    plsc.subcore_barrier()
    bufs = (rows0, rows1)
    sems = (sem0, sem1)
    for b in range(0, nch, IDX_BLK):
        k = min(IDX_BLK, nch - b)
        pltpu.sync_copy(src_hbm.at[wid, pl.ds(b, k)], sidx_v.at[pl.ds(0, k)])
        pltpu.sync_copy(dst_hbm.at[wid, pl.ds(b, k)], didx_v.at[pl.ds(0, k)])
        handles = [pltpu.async_copy(
            featn_hbm.at[sidx_v.at[0]], bufs[0], sems[0])]
        for j in range(k):
            if j + 1 < k:
                handles.append(pltpu.async_copy(
                    featn_hbm.at[sidx_v.at[j + 1]],
                    bufs[(j + 1) % 2], sems[(j + 1) % 2]))
            handles[j].wait()
            pltpu.sync_copy(bufs[j % 2], acc.at[didx_v.at[j]], add=True)
    plsc.subcore_barrier()
    pltpu.sync_copy(acc.at[pl.ds(s * ROWS_PER_TILE, ROWS_PER_TILE)],
                    out_hbm.at[c, pl.ds(s * ROWS_PER_TILE, ROWS_PER_TILE)])


def _norm_scale_body(deg_ref, feat_ref, featn_ref, norm_ref):
    d = deg_ref[0] + deg_ref[1]
    norm = lax.rsqrt(jnp.maximum(d, 1.0))
    norm_ref[...] = norm
    featn_ref[...] = feat_ref[...] * norm


def _out_body(acc_ref, w_ref, norm_ref, bias_ref, out_ref):
    a = acc_ref[0] + acc_ref[1]
    y = jnp.dot(a, w_ref[...], preferred_element_type=jnp.float32)
    out_ref[...] = y * norm_ref[...] + bias_ref[...]


def kernel(feat, edge_index, weight, bias):
    n, d_in = feat.shape
    d_out = weight.shape[1]
    e = edge_index.shape[1]
    nch = -(-e // (NW * CHUNK))
    e_pad = NW * CHUNK * nch

    src = edge_index[0].astype(jnp.int32)
    dst = edge_index[1].astype(jnp.int32)
    pad = jnp.full((e_pad - e,), n, jnp.int32)
    src3 = jnp.concatenate([src, pad]).reshape(NW, nch, CHUNK)
    dst3 = jnp.concatenate([dst, pad]).reshape(NW, nch, CHUNK)

    feat_pad = jnp.zeros((N_PAD, d_in), jnp.float32).at[:n].set(feat)
    zeros2d = jnp.zeros((N_PAD, d_in), jnp.float32)
    zeros1d = jnp.zeros((N_PAD,), jnp.float32)

    mesh = plsc.VectorSubcoreMesh(core_axis_name="c", subcore_axis_name="s")

    deg2 = pl.kernel(
        functools.partial(_deg_body, nch),
        out_type=jax.ShapeDtypeStruct((NC, N_PAD), jnp.float32),
        mesh=mesh,
        scratch_types=[
            pltpu.VMEM((nch, CHUNK), jnp.int32),
            pltpu.VMEM((CHUNK,), jnp.float32),
            pltpu.VMEM_SHARED((N_PAD,), jnp.float32),
            pltpu.SemaphoreType.DMA,
        ],
    )(dst3, zeros1d)

    deg2 = deg2.reshape(NC, N_PAD, 1)

    blk = 1280
    grid = N_PAD // blk
    featn, norm = pl.pallas_call(
        _norm_scale_body,
        grid=(grid,),
        in_specs=[
            pl.BlockSpec((NC, blk, 1), lambda i: (0, i, 0)),
            pl.BlockSpec((blk, d_in), lambda i: (i, 0)),
        ],
        out_specs=[
            pl.BlockSpec((blk, d_in), lambda i: (i, 0)),
            pl.BlockSpec((blk, 1), lambda i: (i, 0)),
        ],
        out_shape=[
            jax.ShapeDtypeStruct((N_PAD, d_in), jnp.float32),
            jax.ShapeDtypeStruct((N_PAD, 1), jnp.float32),
        ],
    )(deg2, feat_pad)

    acc2 = pl.kernel(
        functools.partial(_agg_body, nch),
        out_type=jax.ShapeDtypeStruct((NC, N_PAD, d_in), jnp.float32),
        mesh=mesh,
        scratch_types=[
            pltpu.VMEM((IDX_BLK, CHUNK), jnp.int32),
            pltpu.VMEM((IDX_BLK, CHUNK), jnp.int32),
            pltpu.VMEM((CHUNK, d_in), jnp.float32),
            pltpu.VMEM((CHUNK, d_in), jnp.float32),
            pltpu.VMEM_SHARED((N_PAD, d_in), jnp.float32),
            pltpu.SemaphoreType.DMA,
            pltpu.SemaphoreType.DMA,
        ],
    )(featn, src3, dst3, zeros2d)

    out = pl.pallas_call(
        _out_body,
        grid=(grid,),
        in_specs=[
            pl.BlockSpec((NC, blk, d_in), lambda i: (0, i, 0)),
            pl.BlockSpec((d_in, d_out), lambda i: (0, 0)),
            pl.BlockSpec((blk, 1), lambda i: (i, 0)),
            pl.BlockSpec((1, d_out), lambda i: (0, 0)),
        ],
        out_specs=pl.BlockSpec((blk, d_out), lambda i: (i, 0)),
        out_shape=jax.ShapeDtypeStruct((N_PAD, d_out), jnp.float32),
    )(acc2, weight, norm, bias.reshape(1, d_out))

    return out[:n]


# speedup vs baseline: 7.5475x; 1.3484x over previous
import functools
import jax
import jax.numpy as jnp
from jax import lax
from jax.experimental import pallas as pl
from jax.experimental.pallas import tpu as pltpu
from jax.experimental.pallas import tpu_sc as plsc

N_PAD = 10240
NC = 2
NS = 16
CHUNK = 128
IDX_BLK = 16
ROWS_PER_TILE = N_PAD // NS
SPLIT_FRAC = 0.68


def _split(e):
    t = -(-e // (NS * CHUNK))
    a = min(t, max(1, round(SPLIT_FRAC * t)))
    while NS * a * CHUNK > e:
        a -= 1
    return a, t - a


def _core_guard(fn, c, on_core0_only):
    def run():
        fn()

    if on_core0_only:
        pl.when(c == 0)(run)
    else:
        fn()


def _deg_body(a, b, dst0_hbm, dst1_hbm, zeros_hbm, out_hbm, idx_v, ones_v,
              dacc, sem):
    c = lax.axis_index("c")
    s = lax.axis_index("s")
    for k in range(CHUNK // 16):
        ones_v[pl.ds(k * 16, 16)] = jnp.ones((16,), jnp.float32)
    pltpu.sync_copy(zeros_hbm.at[pl.ds(s * ROWS_PER_TILE, ROWS_PER_TILE)],
                    dacc.at[pl.ds(s * ROWS_PER_TILE, ROWS_PER_TILE)])
    def stage_d0():
        pltpu.sync_copy(dst0_hbm.at[s], idx_v)

    def stage_d1():
        pltpu.sync_copy(dst1_hbm.at[s], idx_v.at[pl.ds(0, b)])

    pl.when(c == 0)(stage_d0)
    pl.when(c == 1)(stage_d1)
    plsc.subcore_barrier()
    for j in range(a):
        _core_guard(
            lambda jj=j: pltpu.sync_copy(ones_v, dacc.at[idx_v.at[jj]],
                                         add=True),
            c, j >= b)
    plsc.subcore_barrier()
    pltpu.sync_copy(dacc.at[pl.ds(s * ROWS_PER_TILE, ROWS_PER_TILE)],
                    out_hbm.at[c, pl.ds(s * ROWS_PER_TILE, ROWS_PER_TILE)])


def _agg_body(a, b, featn_hbm, src0_hbm, src1_hbm, dst0_hbm, dst1_hbm,
              zeros_hbm, out_hbm, sidx_v, didx_v, rows0, rows1, acc,
              sem0, sem1):
    c = lax.axis_index("c")
    s = lax.axis_index("s")
    pltpu.sync_copy(zeros_hbm.at[pl.ds(s * ROWS_PER_TILE, ROWS_PER_TILE)],
                    acc.at[pl.ds(s * ROWS_PER_TILE, ROWS_PER_TILE)])
    plsc.subcore_barrier()
    bufs = (rows0, rows1)
    sems = (sem0, sem1)
    for blk in range(0, a, IDX_BLK):
        k0 = min(IDX_BLK, a - blk)
        def stage0(blk=blk, k0=k0):
            pltpu.sync_copy(src0_hbm.at[s, pl.ds(blk, k0)],
                            sidx_v.at[pl.ds(0, k0)])
            pltpu.sync_copy(dst0_hbm.at[s, pl.ds(blk, k0)],
                            didx_v.at[pl.ds(0, k0)])

        pl.when(c == 0)(stage0)
        if blk < b:
            k1 = min(IDX_BLK, b - blk)

            def stage1(blk=blk, k1=k1):
                pltpu.sync_copy(src1_hbm.at[s, pl.ds(blk, k1)],
                                sidx_v.at[pl.ds(0, k1)])
                pltpu.sync_copy(dst1_hbm.at[s, pl.ds(blk, k1)],
                                didx_v.at[pl.ds(0, k1)])

            pl.when(c == 1)(stage1)
        handles = {}

        def gather(j, g):
            handles[j] = pltpu.async_copy(
                featn_hbm.at[sidx_v.at[j]], bufs[g % 2], sems[g % 2])

        _core_guard(lambda: gather(0, blk), c, blk >= b)
        for j in range(k0):
            g = blk + j
            if j + 1 < k0:
                _core_guard(lambda j=j, g=g: gather(j + 1, g + 1),
                            c, g + 1 >= b)
            _core_guard(lambda j=j: handles[j].wait(), c, g >= b)
            _core_guard(
                lambda j=j, g=g: pltpu.sync_copy(
                    bufs[g % 2], acc.at[didx_v.at[j]], add=True),
                c, g >= b)
    plsc.subcore_barrier()
    pltpu.sync_copy(acc.at[pl.ds(s * ROWS_PER_TILE, ROWS_PER_TILE)],
                    out_hbm.at[c, pl.ds(s * ROWS_PER_TILE, ROWS_PER_TILE)])


def _norm_scale_body(deg_ref, feat_ref, featn_ref, norm_ref):
    d = deg_ref[0] + deg_ref[1]
    norm = lax.rsqrt(jnp.maximum(d, 1.0))
    norm_ref[...] = norm
    featn_ref[...] = feat_ref[...] * norm


def _out_body(acc_ref, w_ref, norm_ref, bias_ref, out_ref):
    a = acc_ref[0] + acc_ref[1]
    y = jnp.dot(a, w_ref[...], preferred_element_type=jnp.float32)
    out_ref[...] = y * norm_ref[...] + bias_ref[...]


def kernel(feat, edge_index, weight, bias):
    n, d_in = feat.shape
    d_out = weight.shape[1]
    e = edge_index.shape[1]
    a, b = _split(e)
    n0 = NS * a * CHUNK
    n1cap = NS * b * CHUNK

    if edge_index.dtype == jnp.int64:
        ei32 = lax.bitcast_convert_type(edge_index, jnp.int32)[..., 0]
    else:
        ei32 = edge_index.astype(jnp.int32)
    src, dst = ei32[0], ei32[1]
    pad = jnp.full((n1cap - (e - n0),), n, jnp.int32)
    src0 = src[:n0].reshape(NS, a, CHUNK)
    dst0 = dst[:n0].reshape(NS, a, CHUNK)
    src1 = jnp.concatenate([src[n0:], pad]).reshape(NS, b, CHUNK)
    dst1 = jnp.concatenate([dst[n0:], pad]).reshape(NS, b, CHUNK)

    zeros2d = jnp.zeros((N_PAD, d_in), jnp.float32)
    zeros1d = jnp.zeros((N_PAD,), jnp.float32)

    mesh = plsc.VectorSubcoreMesh(core_axis_name="c", subcore_axis_name="s")

    deg2 = pl.kernel(
        functools.partial(_deg_body, a, b),
        out_type=jax.ShapeDtypeStruct((NC, N_PAD), jnp.float32),
        mesh=mesh,
        scratch_types=[
            pltpu.VMEM((a, CHUNK), jnp.int32),
            pltpu.VMEM((CHUNK,), jnp.float32),
            pltpu.VMEM_SHARED((N_PAD,), jnp.float32),
            pltpu.SemaphoreType.DMA,
        ],
    )(dst0, dst1, zeros1d)

    deg2 = deg2.reshape(NC, N_PAD, 1)

    blk = 1280
    grid = N_PAD // blk
    featn, norm = pl.pallas_call(
        _norm_scale_body,
        grid=(grid,),
        in_specs=[
            pl.BlockSpec((NC, blk, 1), lambda i: (0, i, 0)),
            pl.BlockSpec((blk, d_in), lambda i: (i, 0)),
        ],
        out_specs=[
            pl.BlockSpec((blk, d_in), lambda i: (i, 0)),
            pl.BlockSpec((blk, 1), lambda i: (i, 0)),
        ],
        out_shape=[
            jax.ShapeDtypeStruct((N_PAD, d_in), jnp.float32),
            jax.ShapeDtypeStruct((N_PAD, 1), jnp.float32),
        ],
    )(deg2, feat)

    acc2 = pl.kernel(
        functools.partial(_agg_body, a, b),
        out_type=jax.ShapeDtypeStruct((NC, N_PAD, d_in), jnp.float32),
        mesh=mesh,
        scratch_types=[
            pltpu.VMEM((IDX_BLK, CHUNK), jnp.int32),
            pltpu.VMEM((IDX_BLK, CHUNK), jnp.int32),
            pltpu.VMEM((CHUNK, d_in), jnp.float32),
            pltpu.VMEM((CHUNK, d_in), jnp.float32),
            pltpu.VMEM_SHARED((N_PAD, d_in), jnp.float32),
            pltpu.SemaphoreType.DMA,
            pltpu.SemaphoreType.DMA,
        ],
    )(featn, src0, src1, dst0, dst1, zeros2d)

    out = pl.pallas_call(
        _out_body,
        grid=(grid,),
        in_specs=[
            pl.BlockSpec((NC, blk, d_in), lambda i: (0, i, 0)),
            pl.BlockSpec((d_in, d_out), lambda i: (0, 0)),
            pl.BlockSpec((blk, 1), lambda i: (i, 0)),
            pl.BlockSpec((1, d_out), lambda i: (0, 0)),
        ],
        out_specs=pl.BlockSpec((blk, d_out), lambda i: (i, 0)),
        out_shape=jax.ShapeDtypeStruct((n, d_out), jnp.float32),
    )(acc2, weight, norm, bias.reshape(1, d_out))

    return out
